# baseline (speedup 1.0000x reference)
import numpy as np
import jax
import jax.numpy as jnp
from jax.experimental import pallas as pl
from jax.experimental.pallas import tpu as pltpu

_TWO_PI = float(2.0 * np.pi)


def _upsample2_linear(traj):
    Nc, L, D = traj.shape
    Lout = 2 * L
    if L == 1:
        return jnp.broadcast_to(traj, (Nc, Lout, D))
    j = jnp.arange(Lout, dtype=jnp.float32)
    pos = j * (L - 1) / (Lout - 1)
    i0 = jnp.clip(jnp.floor(pos).astype(jnp.int32), 0, L - 2)
    frac = pos - i0.astype(jnp.float32)
    lo = traj[:, i0, :]
    hi = traj[:, i0 + 1, :]
    return lo + frac[None, :, None] * (hi - lo)


def _fused_ndft_kernel(G, H, M, W,
                       xr_ref, xi_ref, excw_ref, exsw_ref,
                       excm_ref, exsm_ref, eyc_ref, eys_ref,
                       out_ref, psum_ref):
    f32 = jnp.float32
    xr = xr_ref[...].astype(jnp.bfloat16)
    xi = xi_ref[...].astype(jnp.bfloat16)
    excw = excw_ref[...]
    exsw = exsw_ref[...]

    a_re = (jnp.dot(xr, excw, preferred_element_type=f32)
            + jnp.dot(xi, exsw, preferred_element_type=f32)).reshape(G, H, M)
    a_im = (jnp.dot(xi, excw, preferred_element_type=f32)
            - jnp.dot(xr, exsw, preferred_element_type=f32)).reshape(G, H, M)

    eyc = eyc_ref[...][None]
    eys = eys_ref[...][None]

    ks_re = jnp.sum(a_re * eyc + a_im * eys, axis=1, keepdims=True)
    ks_im = jnp.sum(a_im * eyc - a_re * eys, axis=1, keepdims=True)

    u_re = (ks_re * eyc - ks_im * eys).reshape(G * H, M).astype(jnp.bfloat16)
    u_im = (ks_re * eys + ks_im * eyc).reshape(G * H, M).astype(jnp.bfloat16)

    excm = excm_ref[...]
    exsm = exsm_ref[...]

    adj_re = (jnp.dot(u_re, excm, preferred_element_type=f32)
              - jnp.dot(u_im, exsm, preferred_element_type=f32))
    adj_im = (jnp.dot(u_re, exsm, preferred_element_type=f32)
              + jnp.dot(u_im, excm, preferred_element_type=f32))
    mag = jnp.sqrt(adj_re * adj_re + adj_im * adj_im)
    out_ref[...] = mag
    psum_ref[...] = jnp.sum(mag, axis=0, keepdims=True)[None]


def _forward(x_re, x_im, control):
    B, C, H, W = x_re.shape
    BC = B * C
    R = BC * H

    traj = control
    for _ in range(3):
        traj = _upsample2_linear(traj)
    traj = traj.reshape(-1, traj.shape[-1])
    M = traj.shape[0]

    ax = _TWO_PI * traj[:, 0].astype(jnp.float32)
    ay = _TWO_PI * traj[:, 1].astype(jnp.float32)
    xprime = jnp.arange(W, dtype=jnp.float32) - float(W // 2)
    yprime = jnp.arange(H, dtype=jnp.float32) - float(H // 2)

    phase_x = ax[:, None] * xprime[None, :]
    exc_mw = jnp.cos(phase_x).astype(jnp.bfloat16)
    exs_mw = jnp.sin(phase_x).astype(jnp.bfloat16)
    exc_wm = exc_mw.T
    exs_wm = exs_mw.T

    phase_y = yprime[:, None] * ay[None, :]
    eyc = jnp.cos(phase_y)
    eys = jnp.sin(phase_y)

    xr = x_re.reshape(R, W)
    xi = x_im.reshape(R, W)

    G = 2
    while BC % G != 0 or BC // G < 2:
        G //= 2
        if G == 1:
            break
    rows = G * H
    n_prog = R // rows
    grid = (n_prog,)

    kernel_fn = lambda *refs: _fused_ndft_kernel(G, H, M, W, *refs)

    mag, psum = pl.pallas_call(
        kernel_fn,
        out_shape=(jax.ShapeDtypeStruct((R, W), jnp.float32),
                   jax.ShapeDtypeStruct((n_prog, 1, W), jnp.float32)),
        grid=grid,
        in_specs=[
            pl.BlockSpec((rows, W), lambda i: (i, 0)),
            pl.BlockSpec((rows, W), lambda i: (i, 0)),
            pl.BlockSpec((W, M), lambda i: (0, 0)),
            pl.BlockSpec((W, M), lambda i: (0, 0)),
            pl.BlockSpec((M, W), lambda i: (0, 0)),
            pl.BlockSpec((M, W), lambda i: (0, 0)),
            pl.BlockSpec((H, M), lambda i: (0, 0)),
            pl.BlockSpec((H, M), lambda i: (0, 0)),
        ],
        out_specs=(pl.BlockSpec((rows, W), lambda i: (i, 0)),
                   pl.BlockSpec((1, 1, W), lambda i: (i, 0, 0))),
        compiler_params=pltpu.CompilerParams(
            dimension_semantics=("arbitrary",),
            vmem_limit_bytes=100 * 1024 * 1024),
    )(xr, xi, exc_wm, exs_wm, exc_mw, exs_mw, eyc, eys)

    mean = jnp.sum(psum) / float(R * W)
    out = mag * (1.0 / mean)
    return out.reshape(B, C, H, W)


_forward_jit = jax.jit(_forward)


def kernel(x_re, x_im, control):
    return _forward_jit(x_re, x_im, control)


# speedup vs baseline: 1.1425x; 1.1425x over previous
import numpy as np
import jax
import jax.numpy as jnp
from jax.experimental import pallas as pl
from jax.experimental.pallas import tpu as pltpu

_TWO_PI = float(2.0 * np.pi)


def _upsample2_linear(traj):
    Nc, L, D = traj.shape
    Lout = 2 * L
    if L == 1:
        return jnp.broadcast_to(traj, (Nc, Lout, D))
    j = jnp.arange(Lout, dtype=jnp.float32)
    pos = j * (L - 1) / (Lout - 1)
    i0 = jnp.clip(jnp.floor(pos).astype(jnp.int32), 0, L - 2)
    frac = pos - i0.astype(jnp.float32)
    lo = traj[:, i0, :]
    hi = traj[:, i0 + 1, :]
    return lo + frac[None, :, None] * (hi - lo)


def _fused_ndft_kernel(G, H, M, W,
                       xr_ref, xi_ref, excw_ref, exsw_ref,
                       wadr_ref, wadi_ref, eyc_ref, eys_ref,
                       out_ref, psum_ref):
    f32 = jnp.float32
    xr = xr_ref[...].astype(jnp.bfloat16)
    xi = xi_ref[...].astype(jnp.bfloat16)
    excw = excw_ref[...]
    exsw = exsw_ref[...]

    a_re = (jnp.dot(xr, excw, preferred_element_type=f32)
            + jnp.dot(xi, exsw, preferred_element_type=f32)).reshape(G, H, M)
    a_im = (jnp.dot(xi, excw, preferred_element_type=f32)
            - jnp.dot(xr, exsw, preferred_element_type=f32)).reshape(G, H, M)

    eyc = eyc_ref[...][None]
    eys = eys_ref[...][None]

    ks_re = jnp.sum(a_re * eyc + a_im * eys, axis=1, keepdims=True)
    ks_im = jnp.sum(a_im * eyc - a_re * eys, axis=1, keepdims=True)

    u_re = (ks_re * eyc - ks_im * eys).reshape(G * H, M).astype(jnp.bfloat16)
    u_im = (ks_re * eys + ks_im * eyc).reshape(G * H, M).astype(jnp.bfloat16)
    uc = jnp.concatenate([u_re, u_im], axis=1)

    adj_re = jnp.dot(uc, wadr_ref[...], preferred_element_type=f32)
    adj_im = jnp.dot(uc, wadi_ref[...], preferred_element_type=f32)
    mag = jnp.sqrt(adj_re * adj_re + adj_im * adj_im)
    out_ref[...] = mag.astype(out_ref.dtype)
    psum_ref[...] = jnp.sum(mag, axis=0, keepdims=True)[None]


def _forward(x_re, x_im, control):
    B, C, H, W = x_re.shape
    BC = B * C
    R = BC * H

    traj = control
    for _ in range(3):
        traj = _upsample2_linear(traj)
    traj = traj.reshape(-1, traj.shape[-1])
    M = traj.shape[0]

    ax = _TWO_PI * traj[:, 0].astype(jnp.float32)
    ay = _TWO_PI * traj[:, 1].astype(jnp.float32)
    xprime = jnp.arange(W, dtype=jnp.float32) - float(W // 2)
    yprime = jnp.arange(H, dtype=jnp.float32) - float(H // 2)

    phase_x = ax[:, None] * xprime[None, :]
    exc_mw = jnp.cos(phase_x).astype(jnp.bfloat16)
    exs_mw = jnp.sin(phase_x).astype(jnp.bfloat16)
    exc_wm = exc_mw.T
    exs_wm = exs_mw.T
    wadr = jnp.concatenate([exc_mw, -exs_mw], axis=0)
    wadi = jnp.concatenate([exs_mw, exc_mw], axis=0)

    phase_y = yprime[:, None] * ay[None, :]
    eyc = jnp.cos(phase_y)
    eys = jnp.sin(phase_y)

    xr = x_re.reshape(R, W)
    xi = x_im.reshape(R, W)

    G = 4
    while BC % G != 0 or BC // G < 2:
        G //= 2
        if G == 1:
            break
    rows = G * H
    n_prog = R // rows
    grid = (n_prog,)

    kernel_fn = lambda *refs: _fused_ndft_kernel(G, H, M, W, *refs)

    mag, psum = pl.pallas_call(
        kernel_fn,
        out_shape=(jax.ShapeDtypeStruct((R, W), jnp.bfloat16),
                   jax.ShapeDtypeStruct((n_prog, 1, W), jnp.float32)),
        grid=grid,
        in_specs=[
            pl.BlockSpec((rows, W), lambda i: (i, 0)),
            pl.BlockSpec((rows, W), lambda i: (i, 0)),
            pl.BlockSpec((W, M), lambda i: (0, 0)),
            pl.BlockSpec((W, M), lambda i: (0, 0)),
            pl.BlockSpec((2 * M, W), lambda i: (0, 0)),
            pl.BlockSpec((2 * M, W), lambda i: (0, 0)),
            pl.BlockSpec((H, M), lambda i: (0, 0)),
            pl.BlockSpec((H, M), lambda i: (0, 0)),
        ],
        out_specs=(pl.BlockSpec((rows, W), lambda i: (i, 0)),
                   pl.BlockSpec((1, 1, W), lambda i: (i, 0, 0))),
        compiler_params=pltpu.CompilerParams(
            dimension_semantics=("arbitrary",),
            vmem_limit_bytes=100 * 1024 * 1024),
    )(xr, xi, exc_wm, exs_wm, wadr, wadi, eyc, eys)

    mean = jnp.sum(psum) / float(R * W)
    out = mag.astype(jnp.float32) * (1.0 / mean)
    return out.reshape(B, C, H, W)


_forward_jit = jax.jit(_forward)


def kernel(x_re, x_im, control):
    return _forward_jit(x_re, x_im, control)
